# Initial kernel scaffold; baseline (speedup 1.0000x reference)
import jax
import jax.numpy as jnp
from jax.experimental import pallas as pl


def kernel(h, edge_index, W_self1, W_neigh1, b1, ln1_g, ln1_b, W_self2, W_neigh2, b2, ln2_g, ln2_b, W_self3, W_neigh3, b3):
    raise NotImplementedError("write your pallas kernel here")


# speedup vs baseline: 2.7197x; 2.7197x over previous
import functools

import jax
import jax.numpy as jnp
from jax import lax
from jax.experimental import pallas as pl
from jax.experimental.pallas import tpu as pltpu
from jax.experimental.pallas import tpu_sc as plsc

N = 10000
F_IN = 128
F_HID = 128
F_OUT = 64

NC = 2
NS = 16
NW = NC * NS
K = 128
NCHUNK = 80
G = 4
NGRP = NCHUNK // G
EPAD = NW * NCHUNK * K
NPAD = 10112
CW = 128
CG = 16
RPT = NPAD // NS


def _sc_agg_body(F, y_hbm, src_hbm, dst_hbm, z_hbm,
                 out_hbm, src_v, dst_v, rows_v, acc_sh, sem_g):
    cid = lax.axis_index("c")
    sid = lax.axis_index("s")
    w = cid * NS + sid

    pltpu.sync_copy(z_hbm.at[pl.ds(sid * RPT, RPT)],
                    acc_sh.at[pl.ds(sid * RPT, RPT)])
    plsc.subcore_barrier()

    def group(g, carry):
        pltpu.sync_copy(src_hbm.at[w, pl.ds(g * G, G)], src_v)
        pltpu.sync_copy(dst_hbm.at[w, pl.ds(g * G, G)], dst_v)
        pltpu.async_copy(y_hbm.at[src_v.at[0]], rows_v.at[0], sem_g)
        for j in range(G):
            pltpu.make_async_copy(y_hbm.at[src_v.at[j]], rows_v.at[j % 2],
                                  sem_g).wait()
            if j + 1 < G:
                pltpu.async_copy(y_hbm.at[src_v.at[j + 1]],
                                 rows_v.at[(j + 1) % 2], sem_g)
            pltpu.sync_copy(rows_v.at[j % 2], acc_sh.at[dst_v.at[j]],
                            add=True)
        return carry

    lax.fori_loop(0, NGRP, group, 0)
    plsc.subcore_barrier()

    pltpu.sync_copy(acc_sh.at[pl.ds(sid * RPT, RPT)],
                    out_hbm.at[cid, pl.ds(sid * RPT, RPT)])


@functools.lru_cache(maxsize=None)
def _make_sc_agg(F):
    mesh = plsc.VectorSubcoreMesh(core_axis_name="c", subcore_axis_name="s",
                                  num_cores=NC, num_subcores=NS)
    scratch = [
        pltpu.VMEM((G, K), jnp.int32),
        pltpu.VMEM((G, K), jnp.int32),
        pltpu.VMEM((2, K, F), jnp.float32),
        pltpu.VMEM_SHARED((NPAD, F), jnp.float32),
        pltpu.SemaphoreType.DMA,
    ]
    return pl.kernel(
        functools.partial(_sc_agg_body, F),
        out_type=jax.ShapeDtypeStruct((NC, NPAD, F), jnp.float32),
        mesh=mesh,
        scratch_types=scratch,
    )


def _sc_cnt_body(dst_hbm, ones_hbm, zc_hbm, cnt_hbm,
                 dst_v, ones_v, cnt_sh):
    cid = lax.axis_index("c")
    sid = lax.axis_index("s")
    w = cid * NS + sid

    pltpu.sync_copy(zc_hbm.at[pl.ds(sid * RPT, RPT)],
                    cnt_sh.at[pl.ds(sid * RPT, RPT)])
    pltpu.sync_copy(ones_hbm, ones_v)
    plsc.subcore_barrier()

    def group(g, carry):
        pltpu.sync_copy(dst_hbm.at[w, pl.ds(g * CG, CG)], dst_v)
        for j in range(CG):
            pltpu.sync_copy(ones_v, cnt_sh.at[dst_v.at[j]], add=True)
        return carry

    lax.fori_loop(0, NCHUNK // CG, group, 0)
    plsc.subcore_barrier()

    pltpu.sync_copy(cnt_sh.at[pl.ds(sid * RPT, RPT)],
                    cnt_hbm.at[cid, pl.ds(sid * RPT, RPT)])


@functools.lru_cache(maxsize=None)
def _make_sc_cnt():
    mesh = plsc.VectorSubcoreMesh(core_axis_name="c", subcore_axis_name="s",
                                  num_cores=NC, num_subcores=NS)
    scratch = [
        pltpu.VMEM((CG, K), jnp.int32),
        pltpu.VMEM((K, CW), jnp.float32),
        pltpu.VMEM_SHARED((NPAD, CW), jnp.float32),
    ]
    return pl.kernel(
        _sc_cnt_body,
        out_type=jax.ShapeDtypeStruct((NC, NPAD, CW), jnp.float32),
        mesh=mesh,
        scratch_types=scratch,
    )


def _tc_in_body(h_ref, ws_ref, wn_ref, b_ref, s_ref, n_ref):
    x = h_ref[...]
    s_ref[...] = (jnp.dot(x, ws_ref[...], preferred_element_type=jnp.float32)
                  + b_ref[...])
    n_ref[...] = jnp.dot(x, wn_ref[...], preferred_element_type=jnp.float32)


def _tc_mid_body(s_ref, acc_ref, cnt_ref, g_ref, bn_ref, ws_ref, wn_ref,
                 b_ref, sn_ref, nn_ref):
    deg = cnt_ref[0, :, 0:1] + cnt_ref[1, :, 0:1]
    inv = 1.0 / jnp.maximum(deg, 1.0)
    x = s_ref[...] + (acc_ref[0] + acc_ref[1]) * inv
    mu = jnp.mean(x, axis=-1, keepdims=True)
    var = jnp.mean((x - mu) ** 2, axis=-1, keepdims=True)
    x = (x - mu) * lax.rsqrt(var + 1e-5) * g_ref[...] + bn_ref[...]
    x = jnp.maximum(x, 0.0)
    sn_ref[...] = (jnp.dot(x, ws_ref[...], preferred_element_type=jnp.float32)
                   + b_ref[...])
    nn_ref[...] = jnp.dot(x, wn_ref[...], preferred_element_type=jnp.float32)


def _tc_out_body(s_ref, acc_ref, cnt_ref, o_ref):
    deg = cnt_ref[0, :, 0:1] + cnt_ref[1, :, 0:1]
    inv = 1.0 / jnp.maximum(deg, 1.0)
    o_ref[...] = (s_ref[:, :F_OUT]
                  + (acc_ref[0, :, :F_OUT] + acc_ref[1, :, :F_OUT]) * inv)


_R = 1000


def _full(shape):
    return pl.BlockSpec(shape, lambda i: (0,) * len(shape))


@functools.lru_cache(maxsize=None)
def _make_tc_in(F, FO):
    return pl.pallas_call(
        _tc_in_body,
        grid=(N // _R,),
        in_specs=[
            pl.BlockSpec((_R, F), lambda i: (i, 0)),
            _full((F, FO)), _full((F, FO)), _full((FO,)),
        ],
        out_specs=[pl.BlockSpec((_R, FO), lambda i: (i, 0))] * 2,
        out_shape=[jax.ShapeDtypeStruct((N, FO), jnp.float32)] * 2,
    )


@functools.lru_cache(maxsize=None)
def _make_tc_mid(F, FO):
    return pl.pallas_call(
        _tc_mid_body,
        grid=(N // _R,),
        in_specs=[
            pl.BlockSpec((_R, F), lambda i: (i, 0)),
            pl.BlockSpec((NC, _R, F), lambda i: (0, i, 0)),
            pl.BlockSpec((NC, _R, CW), lambda i: (0, i, 0)),
            _full((F,)), _full((F,)),
            _full((F, FO)), _full((F, FO)), _full((FO,)),
        ],
        out_specs=[pl.BlockSpec((_R, FO), lambda i: (i, 0))] * 2,
        out_shape=[jax.ShapeDtypeStruct((N, FO), jnp.float32)] * 2,
    )


@functools.lru_cache(maxsize=None)
def _make_tc_out():
    return pl.pallas_call(
        _tc_out_body,
        grid=(N // _R,),
        in_specs=[
            pl.BlockSpec((_R, F_HID), lambda i: (i, 0)),
            pl.BlockSpec((NC, _R, F_HID), lambda i: (0, i, 0)),
            pl.BlockSpec((NC, _R, CW), lambda i: (0, i, 0)),
        ],
        out_specs=pl.BlockSpec((_R, F_OUT), lambda i: (i, 0)),
        out_shape=jax.ShapeDtypeStruct((N, F_OUT), jnp.float32),
    )


def kernel(h, edge_index, W_self1, W_neigh1, b1, ln1_g, ln1_b,
           W_self2, W_neigh2, b2, ln2_g, ln2_b, W_self3, W_neigh3, b3):
    src = edge_index[0]
    dst = edge_index[1]
    pad = EPAD - src.shape[0]
    src3 = jnp.concatenate(
        [src, jnp.zeros((pad,), jnp.int32)]).reshape(NW, NCHUNK, K)
    dst3 = jnp.concatenate(
        [dst, jnp.full((pad,), N, jnp.int32)]).reshape(NW, NCHUNK, K)

    zeros_acc = jnp.zeros((NPAD, F_HID), jnp.float32)
    zeros_cnt = jnp.zeros((NPAD, CW), jnp.float32)
    ones_cnt = jnp.ones((K, CW), jnp.float32)

    pad3 = F_HID - F_OUT
    W_self3p = jnp.pad(W_self3, ((0, 0), (0, pad3)))
    W_neigh3p = jnp.pad(W_neigh3, ((0, 0), (0, pad3)))
    b3p = jnp.pad(b3, (0, pad3))

    s1, n1 = _make_tc_in(F_IN, F_HID)(h, W_self1, W_neigh1, b1)
    cnt = _make_sc_cnt()(dst3, ones_cnt, zeros_cnt)
    acc1 = _make_sc_agg(F_HID)(n1, src3, dst3, zeros_acc)
    s2, n2 = _make_tc_mid(F_HID, F_HID)(
        s1, acc1, cnt, ln1_g, ln1_b, W_self2, W_neigh2, b2)
    acc2 = _make_sc_agg(F_HID)(n2, src3, dst3, zeros_acc)
    s3, n3 = _make_tc_mid(F_HID, F_HID)(
        s2, acc2, cnt, ln2_g, ln2_b, W_self3p, W_neigh3p, b3p)
    acc3 = _make_sc_agg(F_HID)(n3, src3, dst3, zeros_acc)
    out = _make_tc_out()(s3, acc3, cnt)
    return out


# speedup vs baseline: 3.2718x; 1.2030x over previous
import functools

import jax
import jax.numpy as jnp
from jax import lax
from jax.experimental import pallas as pl
from jax.experimental.pallas import tpu as pltpu
from jax.experimental.pallas import tpu_sc as plsc

N = 10000
F_IN = 128
F_HID = 128
F_OUT = 64

NC = 2
NS = 16
NW = NC * NS
K = 128
NCHUNK = 80
G = 4
NGRP = NCHUNK // G
EPAD = NW * NCHUNK * K
NPAD = 10112
CW = 128
CG = 16
RPT = NPAD // NS


def _sc_agg_body(F, y_hbm, src_hbm, dst_hbm, z_hbm,
                 out_hbm, src_v, dst_v, rows_v, acc_sh, sem_g):
    cid = lax.axis_index("c")
    sid = lax.axis_index("s")
    w = cid * NS + sid

    pltpu.sync_copy(z_hbm.at[pl.ds(sid * RPT, RPT)],
                    acc_sh.at[pl.ds(sid * RPT, RPT)])
    plsc.subcore_barrier()

    def group(g, carry):
        pltpu.sync_copy(src_hbm.at[w, pl.ds(g * G, G)], src_v)
        pltpu.sync_copy(dst_hbm.at[w, pl.ds(g * G, G)], dst_v)
        pltpu.async_copy(y_hbm.at[src_v.at[0]], rows_v.at[0], sem_g)
        for j in range(G):
            pltpu.make_async_copy(y_hbm.at[src_v.at[j]], rows_v.at[j % 2],
                                  sem_g).wait()
            if j + 1 < G:
                pltpu.async_copy(y_hbm.at[src_v.at[j + 1]],
                                 rows_v.at[(j + 1) % 2], sem_g)
            pltpu.sync_copy(rows_v.at[j % 2], acc_sh.at[dst_v.at[j]],
                            add=True)
        return carry

    lax.fori_loop(0, NGRP, group, 0)
    plsc.subcore_barrier()

    pltpu.sync_copy(acc_sh.at[pl.ds(sid * RPT, RPT)],
                    out_hbm.at[cid, pl.ds(sid * RPT, RPT)])


@functools.lru_cache(maxsize=None)
def _make_sc_agg(F):
    mesh = plsc.VectorSubcoreMesh(core_axis_name="c", subcore_axis_name="s",
                                  num_cores=NC, num_subcores=NS)
    scratch = [
        pltpu.VMEM((G, K), jnp.int32),
        pltpu.VMEM((G, K), jnp.int32),
        pltpu.VMEM((2, K, F), jnp.float32),
        pltpu.VMEM_SHARED((NPAD, F), jnp.float32),
        pltpu.SemaphoreType.DMA,
    ]
    return pl.kernel(
        functools.partial(_sc_agg_body, F),
        out_type=jax.ShapeDtypeStruct((NC, NPAD, F), jnp.float32),
        mesh=mesh,
        scratch_types=scratch,
    )


def _sc_cnt_body(dst_hbm, ones_hbm, zc_hbm, cnt_hbm,
                 dst_v, ones_v, cnt_sh):
    cid = lax.axis_index("c")
    sid = lax.axis_index("s")
    w = cid * NS + sid

    pltpu.sync_copy(zc_hbm.at[pl.ds(sid * RPT, RPT)],
                    cnt_sh.at[pl.ds(sid * RPT, RPT)])
    pltpu.sync_copy(ones_hbm, ones_v)
    plsc.subcore_barrier()

    def group(g, carry):
        pltpu.sync_copy(dst_hbm.at[w, pl.ds(g * CG, CG)], dst_v)
        for j in range(CG):
            pltpu.sync_copy(ones_v, cnt_sh.at[dst_v.at[j]], add=True)
        return carry

    lax.fori_loop(0, NCHUNK // CG, group, 0)
    plsc.subcore_barrier()

    pltpu.sync_copy(cnt_sh.at[pl.ds(sid * RPT, RPT)],
                    cnt_hbm.at[cid, pl.ds(sid * RPT, RPT)])


@functools.lru_cache(maxsize=None)
def _make_sc_cnt():
    mesh = plsc.VectorSubcoreMesh(core_axis_name="c", subcore_axis_name="s",
                                  num_cores=NC, num_subcores=NS)
    scratch = [
        pltpu.VMEM((CG, K), jnp.int32),
        pltpu.VMEM((K, CW), jnp.float32),
        pltpu.VMEM_SHARED((NPAD, CW), jnp.float32),
    ]
    return pl.kernel(
        _sc_cnt_body,
        out_type=jax.ShapeDtypeStruct((NC, NPAD, CW), jnp.float32),
        mesh=mesh,
        scratch_types=scratch,
    )


def _tc_in_body(h_ref, ws_ref, wn_ref, b_ref, s_ref, n_ref):
    x = h_ref[...]
    s_ref[...] = (jnp.dot(x, ws_ref[...], preferred_element_type=jnp.float32)
                  + b_ref[...])
    n_ref[...] = jnp.dot(x, wn_ref[...], preferred_element_type=jnp.float32)


def _tc_mid_body(s_ref, acc_ref, cnt_ref, g_ref, bn_ref, ws_ref, wn_ref,
                 b_ref, sn_ref, nn_ref):
    deg = cnt_ref[0, :, 0:1] + cnt_ref[1, :, 0:1]
    inv = 1.0 / jnp.maximum(deg, 1.0)
    x = s_ref[...] + (acc_ref[0] + acc_ref[1]) * inv
    mu = jnp.mean(x, axis=-1, keepdims=True)
    var = jnp.mean((x - mu) ** 2, axis=-1, keepdims=True)
    x = (x - mu) * lax.rsqrt(var + 1e-5) * g_ref[...] + bn_ref[...]
    x = jnp.maximum(x, 0.0)
    sn_ref[...] = (jnp.dot(x, ws_ref[...], preferred_element_type=jnp.float32)
                   + b_ref[...])
    nn_ref[...] = jnp.dot(x, wn_ref[...], preferred_element_type=jnp.float32)


def _tc_out_body(s_ref, acc_ref, cnt_ref, o_ref):
    deg = cnt_ref[0, :, 0:1] + cnt_ref[1, :, 0:1]
    inv = 1.0 / jnp.maximum(deg, 1.0)
    o_ref[...] = (s_ref[:, :F_OUT]
                  + (acc_ref[0, :, :F_OUT] + acc_ref[1, :, :F_OUT]) * inv)


_R = 1000


def _full(shape):
    return pl.BlockSpec(shape, lambda i: (0,) * len(shape))


@functools.lru_cache(maxsize=None)
def _make_tc_in(F, FO):
    return pl.pallas_call(
        _tc_in_body,
        grid=(N // _R,),
        in_specs=[
            pl.BlockSpec((_R, F), lambda i: (i, 0)),
            _full((F, FO)), _full((F, FO)), _full((FO,)),
        ],
        out_specs=[pl.BlockSpec((_R, FO), lambda i: (i, 0))] * 2,
        out_shape=[jax.ShapeDtypeStruct((N, FO), jnp.float32)] * 2,
    )


@functools.lru_cache(maxsize=None)
def _make_tc_mid(F, FO):
    return pl.pallas_call(
        _tc_mid_body,
        grid=(N // _R,),
        in_specs=[
            pl.BlockSpec((_R, F), lambda i: (i, 0)),
            pl.BlockSpec((NC, _R, F), lambda i: (0, i, 0)),
            pl.BlockSpec((NC, _R, CW), lambda i: (0, i, 0)),
            _full((F,)), _full((F,)),
            _full((F, FO)), _full((F, FO)), _full((FO,)),
        ],
        out_specs=[pl.BlockSpec((_R, FO), lambda i: (i, 0))] * 2,
        out_shape=[jax.ShapeDtypeStruct((N, FO), jnp.float32)] * 2,
    )


@functools.lru_cache(maxsize=None)
def _make_tc_out():
    return pl.pallas_call(
        _tc_out_body,
        grid=(N // _R,),
        in_specs=[
            pl.BlockSpec((_R, F_HID), lambda i: (i, 0)),
            pl.BlockSpec((NC, _R, F_HID), lambda i: (0, i, 0)),
            pl.BlockSpec((NC, _R, CW), lambda i: (0, i, 0)),
        ],
        out_specs=pl.BlockSpec((_R, F_OUT), lambda i: (i, 0)),
        out_shape=jax.ShapeDtypeStruct((N, F_OUT), jnp.float32),
    )


def kernel(h, edge_index, W_self1, W_neigh1, b1, ln1_g, ln1_b,
           W_self2, W_neigh2, b2, ln2_g, ln2_b, W_self3, W_neigh3, b3):
    src = edge_index[0]
    dst = edge_index[1]
    e_per_tile = src.shape[0] // NW
    padt = NCHUNK * K - e_per_tile
    ndummy = NPAD - N
    src3 = jnp.concatenate(
        [src.reshape(NW, e_per_tile),
         jnp.zeros((NW, padt), jnp.int32)], axis=1).reshape(NW, NCHUNK, K)
    dmy = (N + (jnp.arange(NW * padt, dtype=jnp.int32) % ndummy)
           ).reshape(NW, padt)
    dst3 = jnp.concatenate(
        [dst.reshape(NW, e_per_tile), dmy], axis=1).reshape(NW, NCHUNK, K)

    zeros_acc = jnp.zeros((NPAD, F_HID), jnp.float32)
    zeros_cnt = jnp.zeros((NPAD, CW), jnp.float32)
    ones_cnt = jnp.ones((K, CW), jnp.float32)

    pad3 = F_HID - F_OUT
    W_self3p = jnp.pad(W_self3, ((0, 0), (0, pad3)))
    W_neigh3p = jnp.pad(W_neigh3, ((0, 0), (0, pad3)))
    b3p = jnp.pad(b3, (0, pad3))

    s1, n1 = _make_tc_in(F_IN, F_HID)(h, W_self1, W_neigh1, b1)
    cnt = _make_sc_cnt()(dst3, ones_cnt, zeros_cnt)
    acc1 = _make_sc_agg(F_HID)(n1, src3, dst3, zeros_acc)
    s2, n2 = _make_tc_mid(F_HID, F_HID)(
        s1, acc1, cnt, ln1_g, ln1_b, W_self2, W_neigh2, b2)
    acc2 = _make_sc_agg(F_HID)(n2, src3, dst3, zeros_acc)
    s3, n3 = _make_tc_mid(F_HID, F_HID)(
        s2, acc2, cnt, ln2_g, ln2_b, W_self3p, W_neigh3p, b3p)
    acc3 = _make_sc_agg(F_HID)(n3, src3, dst3, zeros_acc)
    out = _make_tc_out()(s3, acc3, cnt)
    return out


# speedup vs baseline: 7.4636x; 2.2812x over previous
import functools

import jax
import jax.numpy as jnp
from jax import lax
from jax.experimental import pallas as pl
from jax.experimental.pallas import tpu as pltpu
from jax.experimental.pallas import tpu_sc as plsc

N = 10000
F_IN = 128
F_HID = 128
F_OUT = 64

NC = 2
NS = 16
NW = NC * NS
K = 128
NCHUNK = 80
G = 4
NGRP = NCHUNK // G
EPAD = NW * NCHUNK * K
NPAD = 10112
CW = 128
CG = 16
RPT = NPAD // NS


def _sc_agg_body(F, y_hbm, src_hbm, dst_hbm, z_hbm,
                 out_hbm, src_v, dst_v, rows_v, acc_sh, sem_g):
    cid = lax.axis_index("c")
    sid = lax.axis_index("s")
    w = cid * NS + sid

    pltpu.sync_copy(z_hbm.at[pl.ds(sid * RPT, RPT)],
                    acc_sh.at[pl.ds(sid * RPT, RPT)])
    plsc.subcore_barrier()

    def group(g, carry):
        pltpu.sync_copy(src_hbm.at[w, pl.ds(g * G, G)], src_v)
        pltpu.sync_copy(dst_hbm.at[w, pl.ds(g * G, G)], dst_v)
        pltpu.async_copy(y_hbm.at[src_v.at[0]], rows_v.at[0], sem_g)
        for j in range(G):
            pltpu.make_async_copy(y_hbm.at[src_v.at[j]], rows_v.at[j % 2],
                                  sem_g).wait()
            if j + 1 < G:
                pltpu.async_copy(y_hbm.at[src_v.at[j + 1]],
                                 rows_v.at[(j + 1) % 2], sem_g)
            pltpu.sync_copy(rows_v.at[j % 2], acc_sh.at[dst_v.at[j]],
                            add=True)
        return carry

    lax.fori_loop(0, NGRP, group, 0)
    plsc.subcore_barrier()

    pltpu.sync_copy(acc_sh.at[pl.ds(sid * RPT, RPT)],
                    out_hbm.at[cid, pl.ds(sid * RPT, RPT)])


@functools.lru_cache(maxsize=None)
def _make_sc_agg(F):
    mesh = plsc.VectorSubcoreMesh(core_axis_name="c", subcore_axis_name="s",
                                  num_cores=NC, num_subcores=NS)
    scratch = [
        pltpu.VMEM((G, K), jnp.int32),
        pltpu.VMEM((G, K), jnp.int32),
        pltpu.VMEM((2, K, F), jnp.float32),
        pltpu.VMEM_SHARED((NPAD, F), jnp.float32),
        pltpu.SemaphoreType.DMA,
    ]
    return pl.kernel(
        functools.partial(_sc_agg_body, F),
        out_type=jax.ShapeDtypeStruct((NC, NPAD, F), jnp.float32),
        mesh=mesh,
        scratch_types=scratch,
    )


def _sc_cnt_body(dst_hbm, ones_hbm, zc_hbm, cnt_hbm,
                 dst_v, ones_v, cnt_sh):
    cid = lax.axis_index("c")
    sid = lax.axis_index("s")
    w = cid * NS + sid

    pltpu.sync_copy(zc_hbm.at[pl.ds(sid * RPT, RPT)],
                    cnt_sh.at[pl.ds(sid * RPT, RPT)])
    pltpu.sync_copy(ones_hbm, ones_v)
    plsc.subcore_barrier()

    def group(g, carry):
        pltpu.sync_copy(dst_hbm.at[w, pl.ds(g * CG, CG)], dst_v)
        for j in range(CG):
            pltpu.sync_copy(ones_v, cnt_sh.at[dst_v.at[j]], add=True)
        return carry

    lax.fori_loop(0, NCHUNK // CG, group, 0)
    plsc.subcore_barrier()

    pltpu.sync_copy(cnt_sh.at[pl.ds(sid * RPT, RPT)],
                    cnt_hbm.at[cid, pl.ds(sid * RPT, RPT)])


@functools.lru_cache(maxsize=None)
def _make_sc_cnt():
    mesh = plsc.VectorSubcoreMesh(core_axis_name="c", subcore_axis_name="s",
                                  num_cores=NC, num_subcores=NS)
    scratch = [
        pltpu.VMEM((CG, K), jnp.int32),
        pltpu.VMEM((K, CW), jnp.float32),
        pltpu.VMEM_SHARED((NPAD, CW), jnp.float32),
    ]
    return pl.kernel(
        _sc_cnt_body,
        out_type=jax.ShapeDtypeStruct((NC, NPAD, CW), jnp.float32),
        mesh=mesh,
        scratch_types=scratch,
    )


def _tc_in_body(h_ref, ws_ref, wn_ref, b_ref, s_ref, n_ref):
    x = h_ref[...]
    s_ref[...] = (jnp.dot(x, ws_ref[...], preferred_element_type=jnp.float32)
                  + b_ref[...])
    n_ref[...] = jnp.dot(x, wn_ref[...], preferred_element_type=jnp.float32)


def _tc_mid_body(s_ref, acc_ref, cnt_ref, g_ref, bn_ref, ws_ref, wn_ref,
                 b_ref, sn_ref, nn_ref):
    deg = cnt_ref[0, :, 0:1] + cnt_ref[1, :, 0:1]
    inv = 1.0 / jnp.maximum(deg, 1.0)
    x = s_ref[...] + (acc_ref[0] + acc_ref[1]) * inv
    mu = jnp.mean(x, axis=-1, keepdims=True)
    var = jnp.mean((x - mu) ** 2, axis=-1, keepdims=True)
    x = (x - mu) * lax.rsqrt(var + 1e-5) * g_ref[...] + bn_ref[...]
    x = jnp.maximum(x, 0.0)
    sn_ref[...] = (jnp.dot(x, ws_ref[...], preferred_element_type=jnp.float32)
                   + b_ref[...])
    nn_ref[...] = jnp.dot(x, wn_ref[...], preferred_element_type=jnp.float32)


def _tc_out_body(s_ref, acc_ref, cnt_ref, o_ref):
    deg = cnt_ref[0, :, 0:1] + cnt_ref[1, :, 0:1]
    inv = 1.0 / jnp.maximum(deg, 1.0)
    o_ref[...] = (s_ref[:, :F_OUT]
                  + (acc_ref[0, :, :F_OUT] + acc_ref[1, :, :F_OUT]) * inv)


_R = 1000


def _full(shape):
    return pl.BlockSpec(shape, lambda i: (0,) * len(shape))


@functools.lru_cache(maxsize=None)
def _make_tc_in(F, FO):
    return pl.pallas_call(
        _tc_in_body,
        grid=(N // _R,),
        in_specs=[
            pl.BlockSpec((_R, F), lambda i: (i, 0)),
            _full((F, FO)), _full((F, FO)), _full((FO,)),
        ],
        out_specs=[pl.BlockSpec((_R, FO), lambda i: (i, 0))] * 2,
        out_shape=[jax.ShapeDtypeStruct((N, FO), jnp.float32)] * 2,
    )


@functools.lru_cache(maxsize=None)
def _make_tc_mid(F, FO):
    return pl.pallas_call(
        _tc_mid_body,
        grid=(N // _R,),
        in_specs=[
            pl.BlockSpec((_R, F), lambda i: (i, 0)),
            pl.BlockSpec((NC, _R, F), lambda i: (0, i, 0)),
            pl.BlockSpec((NC, _R, CW), lambda i: (0, i, 0)),
            _full((F,)), _full((F,)),
            _full((F, FO)), _full((F, FO)), _full((FO,)),
        ],
        out_specs=[pl.BlockSpec((_R, FO), lambda i: (i, 0))] * 2,
        out_shape=[jax.ShapeDtypeStruct((N, FO), jnp.float32)] * 2,
    )


@functools.lru_cache(maxsize=None)
def _make_tc_out():
    return pl.pallas_call(
        _tc_out_body,
        grid=(N // _R,),
        in_specs=[
            pl.BlockSpec((_R, F_HID), lambda i: (i, 0)),
            pl.BlockSpec((NC, _R, F_HID), lambda i: (0, i, 0)),
            pl.BlockSpec((NC, _R, CW), lambda i: (0, i, 0)),
        ],
        out_specs=pl.BlockSpec((_R, F_OUT), lambda i: (i, 0)),
        out_shape=jax.ShapeDtypeStruct((N, F_OUT), jnp.float32),
    )


def kernel(h, edge_index, W_self1, W_neigh1, b1, ln1_g, ln1_b,
           W_self2, W_neigh2, b2, ln2_g, ln2_b, W_self3, W_neigh3, b3):
    src = edge_index[0]
    dst = edge_index[1]
    e_per_tile = src.shape[0] // NW
    padt = NCHUNK * K - e_per_tile
    ndummy = NPAD - N
    psrc = (jnp.arange(NW * padt, dtype=jnp.int32) % N).reshape(NW, padt)
    src3 = jnp.concatenate(
        [src.reshape(NW, e_per_tile), psrc], axis=1).reshape(NW, NCHUNK, K)
    dmy = (N + (jnp.arange(NW * padt, dtype=jnp.int32) % ndummy)
           ).reshape(NW, padt)
    dst3 = jnp.concatenate(
        [dst.reshape(NW, e_per_tile), dmy], axis=1).reshape(NW, NCHUNK, K)

    zeros_acc = jnp.zeros((NPAD, F_HID), jnp.float32)
    zeros_cnt = jnp.zeros((NPAD, CW), jnp.float32)
    ones_cnt = jnp.ones((K, CW), jnp.float32)

    pad3 = F_HID - F_OUT
    W_self3p = jnp.pad(W_self3, ((0, 0), (0, pad3)))
    W_neigh3p = jnp.pad(W_neigh3, ((0, 0), (0, pad3)))
    b3p = jnp.pad(b3, (0, pad3))

    s1, n1 = _make_tc_in(F_IN, F_HID)(h, W_self1, W_neigh1, b1)
    cnt = _make_sc_cnt()(dst3, ones_cnt, zeros_cnt)
    acc1 = _make_sc_agg(F_HID)(n1, src3, dst3, zeros_acc)
    s2, n2 = _make_tc_mid(F_HID, F_HID)(
        s1, acc1, cnt, ln1_g, ln1_b, W_self2, W_neigh2, b2)
    acc2 = _make_sc_agg(F_HID)(n2, src3, dst3, zeros_acc)
    s3, n3 = _make_tc_mid(F_HID, F_HID)(
        s2, acc2, cnt, ln2_g, ln2_b, W_self3p, W_neigh3p, b3p)
    acc3 = _make_sc_agg(F_HID)(n3, src3, dst3, zeros_acc)
    out = _make_tc_out()(s3, acc3, cnt)
    return out


# speedup vs baseline: 7.9888x; 1.0704x over previous
import functools

import jax
import jax.numpy as jnp
from jax import lax
from jax.experimental import pallas as pl
from jax.experimental.pallas import tpu as pltpu
from jax.experimental.pallas import tpu_sc as plsc

N = 10000
F_IN = 128
F_HID = 128
F_OUT = 64

NC = 2
NS = 16
NW = NC * NS
K = 128
NCHUNK = 80
G = 8
NGRP = NCHUNK // G
EPAD = NW * NCHUNK * K
NPAD = 10112
CW = 128
CG = 16
RPT = NPAD // NS


def _sc_agg_body(F, y_hbm, src_hbm, dst_hbm, z_hbm,
                 out_hbm, src_v, dst_v, rows_v, acc_sh, sem_g):
    cid = lax.axis_index("c")
    sid = lax.axis_index("s")
    w = cid * NS + sid

    pltpu.sync_copy(z_hbm.at[pl.ds(sid * RPT, RPT)],
                    acc_sh.at[pl.ds(sid * RPT, RPT)])
    plsc.subcore_barrier()

    def group(g, carry):
        pltpu.sync_copy(src_hbm.at[w, pl.ds(g * G, G)], src_v)
        pltpu.sync_copy(dst_hbm.at[w, pl.ds(g * G, G)], dst_v)
        pltpu.async_copy(y_hbm.at[src_v.at[0]], rows_v.at[0], sem_g)
        for j in range(G):
            pltpu.make_async_copy(y_hbm.at[src_v.at[j]], rows_v.at[j % 2],
                                  sem_g).wait()
            if j + 1 < G:
                pltpu.async_copy(y_hbm.at[src_v.at[j + 1]],
                                 rows_v.at[(j + 1) % 2], sem_g)
            pltpu.sync_copy(rows_v.at[j % 2], acc_sh.at[dst_v.at[j]],
                            add=True)
        return carry

    lax.fori_loop(0, NGRP, group, 0)
    plsc.subcore_barrier()

    pltpu.sync_copy(acc_sh.at[pl.ds(sid * RPT, RPT)],
                    out_hbm.at[cid, pl.ds(sid * RPT, RPT)])


@functools.lru_cache(maxsize=None)
def _make_sc_agg(F):
    mesh = plsc.VectorSubcoreMesh(core_axis_name="c", subcore_axis_name="s",
                                  num_cores=NC, num_subcores=NS)
    scratch = [
        pltpu.VMEM((G, K), jnp.int32),
        pltpu.VMEM((G, K), jnp.int32),
        pltpu.VMEM((2, K, F), jnp.float32),
        pltpu.VMEM_SHARED((NPAD, F), jnp.float32),
        pltpu.SemaphoreType.DMA,
    ]
    return pl.kernel(
        functools.partial(_sc_agg_body, F),
        out_type=jax.ShapeDtypeStruct((NC, NPAD, F), jnp.float32),
        mesh=mesh,
        scratch_types=scratch,
    )


def _sc_cnt_body(dst_hbm, ones_hbm, zc_hbm, cnt_hbm,
                 dst_v, ones_v, cnt_sh):
    cid = lax.axis_index("c")
    sid = lax.axis_index("s")
    w = cid * NS + sid

    pltpu.sync_copy(zc_hbm.at[pl.ds(sid * RPT, RPT)],
                    cnt_sh.at[pl.ds(sid * RPT, RPT)])
    pltpu.sync_copy(ones_hbm, ones_v)
    plsc.subcore_barrier()

    def group(g, carry):
        pltpu.sync_copy(dst_hbm.at[w, pl.ds(g * CG, CG)], dst_v)
        for j in range(CG):
            pltpu.sync_copy(ones_v, cnt_sh.at[dst_v.at[j]], add=True)
        return carry

    lax.fori_loop(0, NCHUNK // CG, group, 0)
    plsc.subcore_barrier()

    pltpu.sync_copy(cnt_sh.at[pl.ds(sid * RPT, RPT)],
                    cnt_hbm.at[cid, pl.ds(sid * RPT, RPT)])


@functools.lru_cache(maxsize=None)
def _make_sc_cnt():
    mesh = plsc.VectorSubcoreMesh(core_axis_name="c", subcore_axis_name="s",
                                  num_cores=NC, num_subcores=NS)
    scratch = [
        pltpu.VMEM((CG, K), jnp.int32),
        pltpu.VMEM((K, CW), jnp.float32),
        pltpu.VMEM_SHARED((NPAD, CW), jnp.float32),
    ]
    return pl.kernel(
        _sc_cnt_body,
        out_type=jax.ShapeDtypeStruct((NC, NPAD, CW), jnp.float32),
        mesh=mesh,
        scratch_types=scratch,
    )


def _tc_in_body(h_ref, ws_ref, wn_ref, b_ref, s_ref, n_ref):
    x = h_ref[...]
    s_ref[...] = (jnp.dot(x, ws_ref[...], preferred_element_type=jnp.float32)
                  + b_ref[...])
    n_ref[...] = jnp.dot(x, wn_ref[...], preferred_element_type=jnp.float32)


def _tc_mid_body(s_ref, acc_ref, cnt_ref, g_ref, bn_ref, ws_ref, wn_ref,
                 b_ref, sn_ref, nn_ref):
    deg = cnt_ref[0, :, 0:1] + cnt_ref[1, :, 0:1]
    inv = 1.0 / jnp.maximum(deg, 1.0)
    x = s_ref[...] + (acc_ref[0] + acc_ref[1]) * inv
    mu = jnp.mean(x, axis=-1, keepdims=True)
    var = jnp.mean((x - mu) ** 2, axis=-1, keepdims=True)
    x = (x - mu) * lax.rsqrt(var + 1e-5) * g_ref[...] + bn_ref[...]
    x = jnp.maximum(x, 0.0)
    sn_ref[...] = (jnp.dot(x, ws_ref[...], preferred_element_type=jnp.float32)
                   + b_ref[...])
    nn_ref[...] = jnp.dot(x, wn_ref[...], preferred_element_type=jnp.float32)


def _tc_out_body(s_ref, acc_ref, cnt_ref, o_ref):
    deg = cnt_ref[0, :, 0:1] + cnt_ref[1, :, 0:1]
    inv = 1.0 / jnp.maximum(deg, 1.0)
    o_ref[...] = (s_ref[:, :F_OUT]
                  + (acc_ref[0, :, :F_OUT] + acc_ref[1, :, :F_OUT]) * inv)


_R = 1000


def _full(shape):
    return pl.BlockSpec(shape, lambda i: (0,) * len(shape))


@functools.lru_cache(maxsize=None)
def _make_tc_in(F, FO):
    return pl.pallas_call(
        _tc_in_body,
        grid=(N // _R,),
        in_specs=[
            pl.BlockSpec((_R, F), lambda i: (i, 0)),
            _full((F, FO)), _full((F, FO)), _full((FO,)),
        ],
        out_specs=[pl.BlockSpec((_R, FO), lambda i: (i, 0))] * 2,
        out_shape=[jax.ShapeDtypeStruct((N, FO), jnp.float32)] * 2,
    )


@functools.lru_cache(maxsize=None)
def _make_tc_mid(F, FO):
    return pl.pallas_call(
        _tc_mid_body,
        grid=(N // _R,),
        in_specs=[
            pl.BlockSpec((_R, F), lambda i: (i, 0)),
            pl.BlockSpec((NC, _R, F), lambda i: (0, i, 0)),
            pl.BlockSpec((NC, _R, CW), lambda i: (0, i, 0)),
            _full((F,)), _full((F,)),
            _full((F, FO)), _full((F, FO)), _full((FO,)),
        ],
        out_specs=[pl.BlockSpec((_R, FO), lambda i: (i, 0))] * 2,
        out_shape=[jax.ShapeDtypeStruct((N, FO), jnp.float32)] * 2,
    )


@functools.lru_cache(maxsize=None)
def _make_tc_out():
    return pl.pallas_call(
        _tc_out_body,
        grid=(N // _R,),
        in_specs=[
            pl.BlockSpec((_R, F_HID), lambda i: (i, 0)),
            pl.BlockSpec((NC, _R, F_HID), lambda i: (0, i, 0)),
            pl.BlockSpec((NC, _R, CW), lambda i: (0, i, 0)),
        ],
        out_specs=pl.BlockSpec((_R, F_OUT), lambda i: (i, 0)),
        out_shape=jax.ShapeDtypeStruct((N, F_OUT), jnp.float32),
    )


def kernel(h, edge_index, W_self1, W_neigh1, b1, ln1_g, ln1_b,
           W_self2, W_neigh2, b2, ln2_g, ln2_b, W_self3, W_neigh3, b3):
    src = edge_index[0]
    dst = edge_index[1]
    e_per_tile = src.shape[0] // NW
    padt = NCHUNK * K - e_per_tile
    ndummy = NPAD - N
    psrc = (jnp.arange(NW * padt, dtype=jnp.int32) % N).reshape(NW, padt)
    src3 = jnp.concatenate(
        [src.reshape(NW, e_per_tile), psrc], axis=1).reshape(NW, NCHUNK, K)
    dmy = (N + (jnp.arange(NW * padt, dtype=jnp.int32) % ndummy)
           ).reshape(NW, padt)
    dst3 = jnp.concatenate(
        [dst.reshape(NW, e_per_tile), dmy], axis=1).reshape(NW, NCHUNK, K)

    zeros_acc = jnp.zeros((NPAD, F_HID), jnp.float32)
    zeros_cnt = jnp.zeros((NPAD, CW), jnp.float32)
    ones_cnt = jnp.ones((K, CW), jnp.float32)

    pad3 = F_HID - F_OUT
    W_self3p = jnp.pad(W_self3, ((0, 0), (0, pad3)))
    W_neigh3p = jnp.pad(W_neigh3, ((0, 0), (0, pad3)))
    b3p = jnp.pad(b3, (0, pad3))

    s1, n1 = _make_tc_in(F_IN, F_HID)(h, W_self1, W_neigh1, b1)
    cnt = _make_sc_cnt()(dst3, ones_cnt, zeros_cnt)
    acc1 = _make_sc_agg(F_HID)(n1, src3, dst3, zeros_acc)
    s2, n2 = _make_tc_mid(F_HID, F_HID)(
        s1, acc1, cnt, ln1_g, ln1_b, W_self2, W_neigh2, b2)
    acc2 = _make_sc_agg(F_HID)(n2, src3, dst3, zeros_acc)
    s3, n3 = _make_tc_mid(F_HID, F_HID)(
        s2, acc2, cnt, ln2_g, ln2_b, W_self3p, W_neigh3p, b3p)
    acc3 = _make_sc_agg(F_HID)(n3, src3, dst3, zeros_acc)
    out = _make_tc_out()(s3, acc3, cnt)
    return out


# speedup vs baseline: 8.2834x; 1.0369x over previous
import functools

import jax
import jax.numpy as jnp
from jax import lax
from jax.experimental import pallas as pl
from jax.experimental.pallas import tpu as pltpu
from jax.experimental.pallas import tpu_sc as plsc

N = 10000
F_IN = 128
F_HID = 128
F_OUT = 64

NC = 2
NS = 16
NW = NC * NS
K = 128
NCHUNK = 80
G = 16
NGRP = NCHUNK // G
EPAD = NW * NCHUNK * K
NPAD = 10112
CW = 128
CG = 16
RPT = NPAD // NS


def _sc_agg_body(F, y_hbm, src_hbm, dst_hbm, z_hbm,
                 out_hbm, src_v, dst_v, rows_v, acc_sh, sem_g):
    cid = lax.axis_index("c")
    sid = lax.axis_index("s")
    w = cid * NS + sid

    pltpu.sync_copy(z_hbm.at[pl.ds(sid * RPT, RPT)],
                    acc_sh.at[pl.ds(sid * RPT, RPT)])
    plsc.subcore_barrier()

    def group(g, carry):
        pltpu.sync_copy(src_hbm.at[w, pl.ds(g * G, G)], src_v)
        pltpu.sync_copy(dst_hbm.at[w, pl.ds(g * G, G)], dst_v)
        pltpu.async_copy(y_hbm.at[src_v.at[0]], rows_v.at[0], sem_g)
        for j in range(G):
            pltpu.make_async_copy(y_hbm.at[src_v.at[j]], rows_v.at[j % 2],
                                  sem_g).wait()
            if j + 1 < G:
                pltpu.async_copy(y_hbm.at[src_v.at[j + 1]],
                                 rows_v.at[(j + 1) % 2], sem_g)
            pltpu.sync_copy(rows_v.at[j % 2], acc_sh.at[dst_v.at[j]],
                            add=True)
        return carry

    lax.fori_loop(0, NGRP, group, 0)
    plsc.subcore_barrier()

    pltpu.sync_copy(acc_sh.at[pl.ds(sid * RPT, RPT)],
                    out_hbm.at[cid, pl.ds(sid * RPT, RPT)])


@functools.lru_cache(maxsize=None)
def _make_sc_agg(F):
    mesh = plsc.VectorSubcoreMesh(core_axis_name="c", subcore_axis_name="s",
                                  num_cores=NC, num_subcores=NS)
    scratch = [
        pltpu.VMEM((G, K), jnp.int32),
        pltpu.VMEM((G, K), jnp.int32),
        pltpu.VMEM((2, K, F), jnp.float32),
        pltpu.VMEM_SHARED((NPAD, F), jnp.float32),
        pltpu.SemaphoreType.DMA,
    ]
    return pl.kernel(
        functools.partial(_sc_agg_body, F),
        out_type=jax.ShapeDtypeStruct((NC, NPAD, F), jnp.float32),
        mesh=mesh,
        scratch_types=scratch,
    )


def _sc_cnt_body(dst_hbm, ones_hbm, zc_hbm, cnt_hbm,
                 dst_v, ones_v, cnt_sh):
    cid = lax.axis_index("c")
    sid = lax.axis_index("s")
    w = cid * NS + sid

    pltpu.sync_copy(zc_hbm.at[pl.ds(sid * RPT, RPT)],
                    cnt_sh.at[pl.ds(sid * RPT, RPT)])
    pltpu.sync_copy(ones_hbm, ones_v)
    plsc.subcore_barrier()

    def group(g, carry):
        pltpu.sync_copy(dst_hbm.at[w, pl.ds(g * CG, CG)], dst_v)
        for j in range(CG):
            pltpu.sync_copy(ones_v, cnt_sh.at[dst_v.at[j]], add=True)
        return carry

    lax.fori_loop(0, NCHUNK // CG, group, 0)
    plsc.subcore_barrier()

    pltpu.sync_copy(cnt_sh.at[pl.ds(sid * RPT, RPT)],
                    cnt_hbm.at[cid, pl.ds(sid * RPT, RPT)])


@functools.lru_cache(maxsize=None)
def _make_sc_cnt():
    mesh = plsc.VectorSubcoreMesh(core_axis_name="c", subcore_axis_name="s",
                                  num_cores=NC, num_subcores=NS)
    scratch = [
        pltpu.VMEM((CG, K), jnp.int32),
        pltpu.VMEM((K, CW), jnp.float32),
        pltpu.VMEM_SHARED((NPAD, CW), jnp.float32),
    ]
    return pl.kernel(
        _sc_cnt_body,
        out_type=jax.ShapeDtypeStruct((NC, NPAD, CW), jnp.float32),
        mesh=mesh,
        scratch_types=scratch,
    )


def _tc_in_body(h_ref, ws_ref, wn_ref, b_ref, s_ref, n_ref):
    x = h_ref[...]
    s_ref[...] = (jnp.dot(x, ws_ref[...], preferred_element_type=jnp.float32)
                  + b_ref[...])
    n_ref[...] = jnp.dot(x, wn_ref[...], preferred_element_type=jnp.float32)


def _tc_mid_body(s_ref, acc_ref, cnt_ref, g_ref, bn_ref, ws_ref, wn_ref,
                 b_ref, sn_ref, nn_ref):
    deg = cnt_ref[0, :, 0:1] + cnt_ref[1, :, 0:1]
    inv = 1.0 / jnp.maximum(deg, 1.0)
    x = s_ref[...] + (acc_ref[0] + acc_ref[1]) * inv
    mu = jnp.mean(x, axis=-1, keepdims=True)
    var = jnp.mean((x - mu) ** 2, axis=-1, keepdims=True)
    x = (x - mu) * lax.rsqrt(var + 1e-5) * g_ref[...] + bn_ref[...]
    x = jnp.maximum(x, 0.0)
    sn_ref[...] = (jnp.dot(x, ws_ref[...], preferred_element_type=jnp.float32)
                   + b_ref[...])
    nn_ref[...] = jnp.dot(x, wn_ref[...], preferred_element_type=jnp.float32)


def _tc_out_body(s_ref, acc_ref, cnt_ref, o_ref):
    deg = cnt_ref[0, :, 0:1] + cnt_ref[1, :, 0:1]
    inv = 1.0 / jnp.maximum(deg, 1.0)
    o_ref[...] = (s_ref[:, :F_OUT]
                  + (acc_ref[0, :, :F_OUT] + acc_ref[1, :, :F_OUT]) * inv)


_R = 1000


def _full(shape):
    return pl.BlockSpec(shape, lambda i: (0,) * len(shape))


@functools.lru_cache(maxsize=None)
def _make_tc_in(F, FO):
    return pl.pallas_call(
        _tc_in_body,
        grid=(N // _R,),
        in_specs=[
            pl.BlockSpec((_R, F), lambda i: (i, 0)),
            _full((F, FO)), _full((F, FO)), _full((FO,)),
        ],
        out_specs=[pl.BlockSpec((_R, FO), lambda i: (i, 0))] * 2,
        out_shape=[jax.ShapeDtypeStruct((N, FO), jnp.float32)] * 2,
    )


@functools.lru_cache(maxsize=None)
def _make_tc_mid(F, FO):
    return pl.pallas_call(
        _tc_mid_body,
        grid=(N // _R,),
        in_specs=[
            pl.BlockSpec((_R, F), lambda i: (i, 0)),
            pl.BlockSpec((NC, _R, F), lambda i: (0, i, 0)),
            pl.BlockSpec((NC, _R, CW), lambda i: (0, i, 0)),
            _full((F,)), _full((F,)),
            _full((F, FO)), _full((F, FO)), _full((FO,)),
        ],
        out_specs=[pl.BlockSpec((_R, FO), lambda i: (i, 0))] * 2,
        out_shape=[jax.ShapeDtypeStruct((N, FO), jnp.float32)] * 2,
    )


@functools.lru_cache(maxsize=None)
def _make_tc_out():
    return pl.pallas_call(
        _tc_out_body,
        grid=(N // _R,),
        in_specs=[
            pl.BlockSpec((_R, F_HID), lambda i: (i, 0)),
            pl.BlockSpec((NC, _R, F_HID), lambda i: (0, i, 0)),
            pl.BlockSpec((NC, _R, CW), lambda i: (0, i, 0)),
        ],
        out_specs=pl.BlockSpec((_R, F_OUT), lambda i: (i, 0)),
        out_shape=jax.ShapeDtypeStruct((N, F_OUT), jnp.float32),
    )


def kernel(h, edge_index, W_self1, W_neigh1, b1, ln1_g, ln1_b,
           W_self2, W_neigh2, b2, ln2_g, ln2_b, W_self3, W_neigh3, b3):
    src = edge_index[0]
    dst = edge_index[1]
    e_per_tile = src.shape[0] // NW
    padt = NCHUNK * K - e_per_tile
    ndummy = NPAD - N
    psrc = (jnp.arange(NW * padt, dtype=jnp.int32) % N).reshape(NW, padt)
    src3 = jnp.concatenate(
        [src.reshape(NW, e_per_tile), psrc], axis=1).reshape(NW, NCHUNK, K)
    dmy = (N + (jnp.arange(NW * padt, dtype=jnp.int32) % ndummy)
           ).reshape(NW, padt)
    dst3 = jnp.concatenate(
        [dst.reshape(NW, e_per_tile), dmy], axis=1).reshape(NW, NCHUNK, K)

    zeros_acc = jnp.zeros((NPAD, F_HID), jnp.float32)
    zeros_cnt = jnp.zeros((NPAD, CW), jnp.float32)
    ones_cnt = jnp.ones((K, CW), jnp.float32)

    pad3 = F_HID - F_OUT
    W_self3p = jnp.pad(W_self3, ((0, 0), (0, pad3)))
    W_neigh3p = jnp.pad(W_neigh3, ((0, 0), (0, pad3)))
    b3p = jnp.pad(b3, (0, pad3))

    s1, n1 = _make_tc_in(F_IN, F_HID)(h, W_self1, W_neigh1, b1)
    cnt = _make_sc_cnt()(dst3, ones_cnt, zeros_cnt)
    acc1 = _make_sc_agg(F_HID)(n1, src3, dst3, zeros_acc)
    s2, n2 = _make_tc_mid(F_HID, F_HID)(
        s1, acc1, cnt, ln1_g, ln1_b, W_self2, W_neigh2, b2)
    acc2 = _make_sc_agg(F_HID)(n2, src3, dst3, zeros_acc)
    s3, n3 = _make_tc_mid(F_HID, F_HID)(
        s2, acc2, cnt, ln2_g, ln2_b, W_self3p, W_neigh3p, b3p)
    acc3 = _make_sc_agg(F_HID)(n3, src3, dst3, zeros_acc)
    out = _make_tc_out()(s3, acc3, cnt)
    return out


# speedup vs baseline: 8.5268x; 1.0294x over previous
import functools

import jax
import jax.numpy as jnp
from jax import lax
from jax.experimental import pallas as pl
from jax.experimental.pallas import tpu as pltpu
from jax.experimental.pallas import tpu_sc as plsc

N = 10000
F_IN = 128
F_HID = 128
F_OUT = 64

NC = 2
NS = 16
NW = NC * NS
K = 128
NCHUNK = 80
G = 40
NGRP = NCHUNK // G
EPAD = NW * NCHUNK * K
NPAD = 10112
CW = 128
CG = 16
RPT = NPAD // NS


def _sc_agg_body(F, y_hbm, src_hbm, dst_hbm, z_hbm,
                 out_hbm, src_v, dst_v, rows_v, acc_sh, sem_g):
    cid = lax.axis_index("c")
    sid = lax.axis_index("s")
    w = cid * NS + sid

    pltpu.sync_copy(z_hbm.at[pl.ds(sid * RPT, RPT)],
                    acc_sh.at[pl.ds(sid * RPT, RPT)])
    plsc.subcore_barrier()

    def group(g, carry):
        pltpu.sync_copy(src_hbm.at[w, pl.ds(g * G, G)], src_v)
        pltpu.sync_copy(dst_hbm.at[w, pl.ds(g * G, G)], dst_v)
        pltpu.async_copy(y_hbm.at[src_v.at[0]], rows_v.at[0], sem_g)
        for j in range(G):
            pltpu.make_async_copy(y_hbm.at[src_v.at[j]], rows_v.at[j % 2],
                                  sem_g).wait()
            if j + 1 < G:
                pltpu.async_copy(y_hbm.at[src_v.at[j + 1]],
                                 rows_v.at[(j + 1) % 2], sem_g)
            pltpu.sync_copy(rows_v.at[j % 2], acc_sh.at[dst_v.at[j]],
                            add=True)
        return carry

    lax.fori_loop(0, NGRP, group, 0)
    plsc.subcore_barrier()

    pltpu.sync_copy(acc_sh.at[pl.ds(sid * RPT, RPT)],
                    out_hbm.at[cid, pl.ds(sid * RPT, RPT)])


@functools.lru_cache(maxsize=None)
def _make_sc_agg(F):
    mesh = plsc.VectorSubcoreMesh(core_axis_name="c", subcore_axis_name="s",
                                  num_cores=NC, num_subcores=NS)
    scratch = [
        pltpu.VMEM((G, K), jnp.int32),
        pltpu.VMEM((G, K), jnp.int32),
        pltpu.VMEM((2, K, F), jnp.float32),
        pltpu.VMEM_SHARED((NPAD, F), jnp.float32),
        pltpu.SemaphoreType.DMA,
    ]
    return pl.kernel(
        functools.partial(_sc_agg_body, F),
        out_type=jax.ShapeDtypeStruct((NC, NPAD, F), jnp.float32),
        mesh=mesh,
        scratch_types=scratch,
    )


def _sc_cnt_body(dst_hbm, ones_hbm, zc_hbm, cnt_hbm,
                 dst_v, ones_v, cnt_sh):
    cid = lax.axis_index("c")
    sid = lax.axis_index("s")
    w = cid * NS + sid

    pltpu.sync_copy(zc_hbm.at[pl.ds(sid * RPT, RPT)],
                    cnt_sh.at[pl.ds(sid * RPT, RPT)])
    pltpu.sync_copy(ones_hbm, ones_v)
    plsc.subcore_barrier()

    def group(g, carry):
        pltpu.sync_copy(dst_hbm.at[w, pl.ds(g * CG, CG)], dst_v)
        for j in range(CG):
            pltpu.sync_copy(ones_v, cnt_sh.at[dst_v.at[j]], add=True)
        return carry

    lax.fori_loop(0, NCHUNK // CG, group, 0)
    plsc.subcore_barrier()

    pltpu.sync_copy(cnt_sh.at[pl.ds(sid * RPT, RPT)],
                    cnt_hbm.at[cid, pl.ds(sid * RPT, RPT)])


@functools.lru_cache(maxsize=None)
def _make_sc_cnt():
    mesh = plsc.VectorSubcoreMesh(core_axis_name="c", subcore_axis_name="s",
                                  num_cores=NC, num_subcores=NS)
    scratch = [
        pltpu.VMEM((CG, K), jnp.int32),
        pltpu.VMEM((K, CW), jnp.float32),
        pltpu.VMEM_SHARED((NPAD, CW), jnp.float32),
    ]
    return pl.kernel(
        _sc_cnt_body,
        out_type=jax.ShapeDtypeStruct((NC, NPAD, CW), jnp.float32),
        mesh=mesh,
        scratch_types=scratch,
    )


def _tc_in_body(h_ref, ws_ref, wn_ref, b_ref, s_ref, n_ref):
    x = h_ref[...]
    s_ref[...] = (jnp.dot(x, ws_ref[...], preferred_element_type=jnp.float32)
                  + b_ref[...])
    n_ref[...] = jnp.dot(x, wn_ref[...], preferred_element_type=jnp.float32)


def _tc_mid_body(s_ref, acc_ref, cnt_ref, g_ref, bn_ref, ws_ref, wn_ref,
                 b_ref, sn_ref, nn_ref):
    deg = cnt_ref[0, :, 0:1] + cnt_ref[1, :, 0:1]
    inv = 1.0 / jnp.maximum(deg, 1.0)
    x = s_ref[...] + (acc_ref[0] + acc_ref[1]) * inv
    mu = jnp.mean(x, axis=-1, keepdims=True)
    var = jnp.mean((x - mu) ** 2, axis=-1, keepdims=True)
    x = (x - mu) * lax.rsqrt(var + 1e-5) * g_ref[...] + bn_ref[...]
    x = jnp.maximum(x, 0.0)
    sn_ref[...] = (jnp.dot(x, ws_ref[...], preferred_element_type=jnp.float32)
                   + b_ref[...])
    nn_ref[...] = jnp.dot(x, wn_ref[...], preferred_element_type=jnp.float32)


def _tc_out_body(s_ref, acc_ref, cnt_ref, o_ref):
    deg = cnt_ref[0, :, 0:1] + cnt_ref[1, :, 0:1]
    inv = 1.0 / jnp.maximum(deg, 1.0)
    o_ref[...] = (s_ref[:, :F_OUT]
                  + (acc_ref[0, :, :F_OUT] + acc_ref[1, :, :F_OUT]) * inv)


_R = 1000


def _full(shape):
    return pl.BlockSpec(shape, lambda i: (0,) * len(shape))


@functools.lru_cache(maxsize=None)
def _make_tc_in(F, FO):
    return pl.pallas_call(
        _tc_in_body,
        grid=(N // _R,),
        in_specs=[
            pl.BlockSpec((_R, F), lambda i: (i, 0)),
            _full((F, FO)), _full((F, FO)), _full((FO,)),
        ],
        out_specs=[pl.BlockSpec((_R, FO), lambda i: (i, 0))] * 2,
        out_shape=[jax.ShapeDtypeStruct((N, FO), jnp.float32)] * 2,
    )


@functools.lru_cache(maxsize=None)
def _make_tc_mid(F, FO):
    return pl.pallas_call(
        _tc_mid_body,
        grid=(N // _R,),
        in_specs=[
            pl.BlockSpec((_R, F), lambda i: (i, 0)),
            pl.BlockSpec((NC, _R, F), lambda i: (0, i, 0)),
            pl.BlockSpec((NC, _R, CW), lambda i: (0, i, 0)),
            _full((F,)), _full((F,)),
            _full((F, FO)), _full((F, FO)), _full((FO,)),
        ],
        out_specs=[pl.BlockSpec((_R, FO), lambda i: (i, 0))] * 2,
        out_shape=[jax.ShapeDtypeStruct((N, FO), jnp.float32)] * 2,
    )


@functools.lru_cache(maxsize=None)
def _make_tc_out():
    return pl.pallas_call(
        _tc_out_body,
        grid=(N // _R,),
        in_specs=[
            pl.BlockSpec((_R, F_HID), lambda i: (i, 0)),
            pl.BlockSpec((NC, _R, F_HID), lambda i: (0, i, 0)),
            pl.BlockSpec((NC, _R, CW), lambda i: (0, i, 0)),
        ],
        out_specs=pl.BlockSpec((_R, F_OUT), lambda i: (i, 0)),
        out_shape=jax.ShapeDtypeStruct((N, F_OUT), jnp.float32),
    )


def kernel(h, edge_index, W_self1, W_neigh1, b1, ln1_g, ln1_b,
           W_self2, W_neigh2, b2, ln2_g, ln2_b, W_self3, W_neigh3, b3):
    src = edge_index[0]
    dst = edge_index[1]
    e_per_tile = src.shape[0] // NW
    padt = NCHUNK * K - e_per_tile
    ndummy = NPAD - N
    psrc = (jnp.arange(NW * padt, dtype=jnp.int32) % N).reshape(NW, padt)
    src3 = jnp.concatenate(
        [src.reshape(NW, e_per_tile), psrc], axis=1).reshape(NW, NCHUNK, K)
    dmy = (N + (jnp.arange(NW * padt, dtype=jnp.int32) % ndummy)
           ).reshape(NW, padt)
    dst3 = jnp.concatenate(
        [dst.reshape(NW, e_per_tile), dmy], axis=1).reshape(NW, NCHUNK, K)

    zeros_acc = jnp.zeros((NPAD, F_HID), jnp.float32)
    zeros_cnt = jnp.zeros((NPAD, CW), jnp.float32)
    ones_cnt = jnp.ones((K, CW), jnp.float32)

    pad3 = F_HID - F_OUT
    W_self3p = jnp.pad(W_self3, ((0, 0), (0, pad3)))
    W_neigh3p = jnp.pad(W_neigh3, ((0, 0), (0, pad3)))
    b3p = jnp.pad(b3, (0, pad3))

    s1, n1 = _make_tc_in(F_IN, F_HID)(h, W_self1, W_neigh1, b1)
    cnt = _make_sc_cnt()(dst3, ones_cnt, zeros_cnt)
    acc1 = _make_sc_agg(F_HID)(n1, src3, dst3, zeros_acc)
    s2, n2 = _make_tc_mid(F_HID, F_HID)(
        s1, acc1, cnt, ln1_g, ln1_b, W_self2, W_neigh2, b2)
    acc2 = _make_sc_agg(F_HID)(n2, src3, dst3, zeros_acc)
    s3, n3 = _make_tc_mid(F_HID, F_HID)(
        s2, acc2, cnt, ln2_g, ln2_b, W_self3p, W_neigh3p, b3p)
    acc3 = _make_sc_agg(F_HID)(n3, src3, dst3, zeros_acc)
    out = _make_tc_out()(s3, acc3, cnt)
    return out


# speedup vs baseline: 8.6606x; 1.0157x over previous
import functools

import jax
import jax.numpy as jnp
from jax import lax
from jax.experimental import pallas as pl
from jax.experimental.pallas import tpu as pltpu
from jax.experimental.pallas import tpu_sc as plsc

N = 10000
F_IN = 128
F_HID = 128
F_OUT = 64

NC = 2
NS = 16
NW = NC * NS
K = 128
NCHUNK = 80
G = 40
NGRP = NCHUNK // G
EPAD = NW * NCHUNK * K
NPAD = 10112
CW = 128
CG = 16
RPT = NPAD // NS


def _sc_agg_body(F, y_hbm, src_hbm, dst_hbm, z_hbm,
                 out_hbm, src_v, dst_v, rows_v, acc_sh, sem_g):
    cid = lax.axis_index("c")
    sid = lax.axis_index("s")
    w = cid * NS + sid

    pltpu.sync_copy(z_hbm.at[pl.ds(sid * RPT, RPT)],
                    acc_sh.at[pl.ds(sid * RPT, RPT)])
    plsc.subcore_barrier()

    def group(g, carry):
        pltpu.sync_copy(src_hbm.at[w, pl.ds(g * G, G)], src_v)
        pltpu.sync_copy(dst_hbm.at[w, pl.ds(g * G, G)], dst_v)
        pltpu.async_copy(y_hbm.at[src_v.at[0]], rows_v.at[0], sem_g)
        for j in range(G):
            pltpu.make_async_copy(y_hbm.at[src_v.at[j]], rows_v.at[j % 2],
                                  sem_g).wait()
            if j + 1 < G:
                pltpu.async_copy(y_hbm.at[src_v.at[j + 1]],
                                 rows_v.at[(j + 1) % 2], sem_g)
            pltpu.sync_copy(rows_v.at[j % 2], acc_sh.at[dst_v.at[j]],
                            add=True)
        return carry

    lax.fori_loop(0, NGRP, group, 0)
    plsc.subcore_barrier()

    pltpu.sync_copy(acc_sh.at[pl.ds(sid * RPT, RPT)],
                    out_hbm.at[cid, pl.ds(sid * RPT, RPT)])


@functools.lru_cache(maxsize=None)
def _make_sc_agg(F):
    mesh = plsc.VectorSubcoreMesh(core_axis_name="c", subcore_axis_name="s",
                                  num_cores=NC, num_subcores=NS)
    scratch = [
        pltpu.VMEM((G, K), jnp.int32),
        pltpu.VMEM((G, K), jnp.int32),
        pltpu.VMEM((2, K, F), jnp.float32),
        pltpu.VMEM_SHARED((NPAD, F), jnp.float32),
        pltpu.SemaphoreType.DMA,
    ]
    return pl.kernel(
        functools.partial(_sc_agg_body, F),
        out_type=jax.ShapeDtypeStruct((NC, NPAD, F), jnp.float32),
        mesh=mesh,
        scratch_types=scratch,
    )


def _sc_cnt_body(dst_hbm, ones_hbm, zc_hbm, cnt_hbm,
                 dst_v, ones_v, cnt_sh):
    cid = lax.axis_index("c")
    sid = lax.axis_index("s")
    w = cid * NS + sid

    pltpu.sync_copy(zc_hbm.at[pl.ds(sid * RPT, RPT)],
                    cnt_sh.at[pl.ds(sid * RPT, RPT)])
    pltpu.sync_copy(ones_hbm, ones_v)
    plsc.subcore_barrier()

    def group(g, carry):
        pltpu.sync_copy(dst_hbm.at[w, pl.ds(g * CG, CG)], dst_v)
        for j in range(CG):
            pltpu.sync_copy(ones_v, cnt_sh.at[dst_v.at[j]], add=True)
        return carry

    lax.fori_loop(0, NCHUNK // CG, group, 0)
    plsc.subcore_barrier()

    pltpu.sync_copy(cnt_sh.at[pl.ds(sid * RPT, RPT)],
                    cnt_hbm.at[cid, pl.ds(sid * RPT, RPT)])


@functools.lru_cache(maxsize=None)
def _make_sc_cnt():
    mesh = plsc.VectorSubcoreMesh(core_axis_name="c", subcore_axis_name="s",
                                  num_cores=NC, num_subcores=NS)
    scratch = [
        pltpu.VMEM((CG, K), jnp.int32),
        pltpu.VMEM((K, CW), jnp.float32),
        pltpu.VMEM_SHARED((NPAD, CW), jnp.float32),
    ]
    return pl.kernel(
        _sc_cnt_body,
        out_type=jax.ShapeDtypeStruct((NC, NPAD, CW), jnp.float32),
        mesh=mesh,
        scratch_types=scratch,
    )


def _tc_in_body(h_ref, ws_ref, wn_ref, b_ref, s_ref, n_ref):
    x = h_ref[...]
    s_ref[...] = (jnp.dot(x, ws_ref[...], preferred_element_type=jnp.float32)
                  + b_ref[...])
    n_ref[...] = jnp.dot(x, wn_ref[...], preferred_element_type=jnp.float32)


def _tc_mid_body(s_ref, acc_ref, cnt_ref, g_ref, bn_ref, ws_ref, wn_ref,
                 b_ref, sn_ref, nn_ref):
    deg = cnt_ref[0, :, 0:1] + cnt_ref[1, :, 0:1]
    inv = 1.0 / jnp.maximum(deg, 1.0)
    x = s_ref[...] + (acc_ref[0] + acc_ref[1]) * inv
    mu = jnp.mean(x, axis=-1, keepdims=True)
    var = jnp.mean((x - mu) ** 2, axis=-1, keepdims=True)
    x = (x - mu) * lax.rsqrt(var + 1e-5) * g_ref[...] + bn_ref[...]
    x = jnp.maximum(x, 0.0)
    sn_ref[...] = (jnp.dot(x, ws_ref[...], preferred_element_type=jnp.float32)
                   + b_ref[...])
    nn_ref[...] = jnp.dot(x, wn_ref[...], preferred_element_type=jnp.float32)


def _tc_out_body(s_ref, acc_ref, cnt_ref, o_ref):
    deg = cnt_ref[0, :, 0:1] + cnt_ref[1, :, 0:1]
    inv = 1.0 / jnp.maximum(deg, 1.0)
    o_ref[...] = (s_ref[:, :F_OUT]
                  + (acc_ref[0, :, :F_OUT] + acc_ref[1, :, :F_OUT]) * inv)


_R = 1000


def _full(shape):
    return pl.BlockSpec(shape, lambda i: (0,) * len(shape))


@functools.lru_cache(maxsize=None)
def _make_tc_in(F, FO):
    return pl.pallas_call(
        _tc_in_body,
        grid=(N // _R,),
        in_specs=[
            pl.BlockSpec((_R, F), lambda i: (i, 0)),
            _full((F, FO)), _full((F, FO)), _full((FO,)),
        ],
        out_specs=[pl.BlockSpec((_R, FO), lambda i: (i, 0))] * 2,
        out_shape=[jax.ShapeDtypeStruct((N, FO), jnp.float32)] * 2,
    )


@functools.lru_cache(maxsize=None)
def _make_tc_mid(F, FO):
    return pl.pallas_call(
        _tc_mid_body,
        grid=(N // _R,),
        in_specs=[
            pl.BlockSpec((_R, F), lambda i: (i, 0)),
            pl.BlockSpec((NC, _R, F), lambda i: (0, i, 0)),
            pl.BlockSpec((NC, _R, CW), lambda i: (0, i, 0)),
            _full((F,)), _full((F,)),
            _full((F, FO)), _full((F, FO)), _full((FO,)),
        ],
        out_specs=[pl.BlockSpec((_R, FO), lambda i: (i, 0))] * 2,
        out_shape=[jax.ShapeDtypeStruct((N, FO), jnp.float32)] * 2,
    )


@functools.lru_cache(maxsize=None)
def _make_tc_out():
    return pl.pallas_call(
        _tc_out_body,
        grid=(N // _R,),
        in_specs=[
            pl.BlockSpec((_R, F_HID), lambda i: (i, 0)),
            pl.BlockSpec((NC, _R, F_HID), lambda i: (0, i, 0)),
            pl.BlockSpec((NC, _R, CW), lambda i: (0, i, 0)),
        ],
        out_specs=pl.BlockSpec((_R, F_OUT), lambda i: (i, 0)),
        out_shape=jax.ShapeDtypeStruct((N, F_OUT), jnp.float32),
    )


def kernel(h, edge_index, W_self1, W_neigh1, b1, ln1_g, ln1_b,
           W_self2, W_neigh2, b2, ln2_g, ln2_b, W_self3, W_neigh3, b3):
    src = edge_index[0]
    dst = edge_index[1]
    e_per_tile = src.shape[0] // NW
    padt = NCHUNK * K - e_per_tile
    ndummy = NPAD - N
    psrc = (jnp.arange(NW * padt, dtype=jnp.int32) % N).reshape(NW, padt)
    src3 = jnp.concatenate(
        [src.reshape(NW, e_per_tile), psrc], axis=1).reshape(NW, NCHUNK, K)
    dmy = (N + (jnp.arange(NW * padt, dtype=jnp.int32) % ndummy)
           ).reshape(NW, padt)
    dst3 = jnp.concatenate(
        [dst.reshape(NW, e_per_tile), dmy], axis=1).reshape(NW, NCHUNK, K)

    zeros_acc = jnp.zeros((NPAD, F_HID), jnp.float32)
    zeros_cnt = jnp.zeros((NPAD, CW), jnp.float32)
    ones_cnt = jnp.ones((K, CW), jnp.float32)

    pad3 = F_HID - F_OUT
    W_self3p = jnp.pad(W_self3, ((0, 0), (0, pad3)))
    W_neigh3p = jnp.pad(W_neigh3, ((0, 0), (0, pad3)))
    b3p = jnp.pad(b3, (0, pad3))

    s1, n1 = _make_tc_in(F_IN, F_HID)(h, W_self1, W_neigh1, b1)
    cnt = _make_sc_cnt()(dst3, ones_cnt, zeros_cnt)
    tok = (cnt[0, 0, 0] * 0.0).astype(jnp.int32)
    acc1 = _make_sc_agg(F_HID)(n1, src3 + tok, dst3, zeros_acc)
    s2, n2 = _make_tc_mid(F_HID, F_HID)(
        s1, acc1, cnt, ln1_g, ln1_b, W_self2, W_neigh2, b2)
    acc2 = _make_sc_agg(F_HID)(n2, src3, dst3, zeros_acc)
    s3, n3 = _make_tc_mid(F_HID, F_HID)(
        s2, acc2, cnt, ln2_g, ln2_b, W_self3p, W_neigh3p, b3p)
    acc3 = _make_sc_agg(F_HID)(n3, src3, dst3, zeros_acc)
    out = _make_tc_out()(s3, acc3, cnt)
    return out


# speedup vs baseline: 8.6879x; 1.0032x over previous
import functools

import jax
import jax.numpy as jnp
from jax import lax
from jax.experimental import pallas as pl
from jax.experimental.pallas import tpu as pltpu
from jax.experimental.pallas import tpu_sc as plsc

N = 10000
F_IN = 128
F_HID = 128
F_OUT = 64

NC = 2
NS = 16
NW = NC * NS
K = 128
NCHUNK = 80
G = 40
NGRP = NCHUNK // G
EPAD = NW * NCHUNK * K
NPAD = 10112
CW = 128
CG = 40
RPT = NPAD // NS


def _sc_agg_body(F, y_hbm, src_hbm, dst_hbm, z_hbm,
                 out_hbm, src_v, dst_v, rows_v, acc_sh, sem_g):
    cid = lax.axis_index("c")
    sid = lax.axis_index("s")
    w = cid * NS + sid

    pltpu.sync_copy(z_hbm.at[pl.ds(sid * RPT, RPT)],
                    acc_sh.at[pl.ds(sid * RPT, RPT)])
    plsc.subcore_barrier()

    def group(g, carry):
        pltpu.sync_copy(src_hbm.at[w, pl.ds(g * G, G)], src_v)
        pltpu.sync_copy(dst_hbm.at[w, pl.ds(g * G, G)], dst_v)
        pltpu.async_copy(y_hbm.at[src_v.at[0]], rows_v.at[0], sem_g)
        for j in range(G):
            pltpu.make_async_copy(y_hbm.at[src_v.at[j]], rows_v.at[j % 2],
                                  sem_g).wait()
            if j + 1 < G:
                pltpu.async_copy(y_hbm.at[src_v.at[j + 1]],
                                 rows_v.at[(j + 1) % 2], sem_g)
            pltpu.sync_copy(rows_v.at[j % 2], acc_sh.at[dst_v.at[j]],
                            add=True)
        return carry

    lax.fori_loop(0, NGRP, group, 0)
    plsc.subcore_barrier()

    pltpu.sync_copy(acc_sh.at[pl.ds(sid * RPT, RPT)],
                    out_hbm.at[cid, pl.ds(sid * RPT, RPT)])


@functools.lru_cache(maxsize=None)
def _make_sc_agg(F):
    mesh = plsc.VectorSubcoreMesh(core_axis_name="c", subcore_axis_name="s",
                                  num_cores=NC, num_subcores=NS)
    scratch = [
        pltpu.VMEM((G, K), jnp.int32),
        pltpu.VMEM((G, K), jnp.int32),
        pltpu.VMEM((2, K, F), jnp.float32),
        pltpu.VMEM_SHARED((NPAD, F), jnp.float32),
        pltpu.SemaphoreType.DMA,
    ]
    return pl.kernel(
        functools.partial(_sc_agg_body, F),
        out_type=jax.ShapeDtypeStruct((NC, NPAD, F), jnp.float32),
        mesh=mesh,
        scratch_types=scratch,
    )


def _sc_cnt_body(dst_hbm, ones_hbm, zc_hbm, cnt_hbm,
                 dst_v, ones_v, cnt_sh):
    cid = lax.axis_index("c")
    sid = lax.axis_index("s")
    w = cid * NS + sid

    pltpu.sync_copy(zc_hbm.at[pl.ds(sid * RPT, RPT)],
                    cnt_sh.at[pl.ds(sid * RPT, RPT)])
    pltpu.sync_copy(ones_hbm, ones_v)
    plsc.subcore_barrier()

    def group(g, carry):
        pltpu.sync_copy(dst_hbm.at[w, pl.ds(g * CG, CG)], dst_v)
        for j in range(CG):
            pltpu.sync_copy(ones_v, cnt_sh.at[dst_v.at[j]], add=True)
        return carry

    lax.fori_loop(0, NCHUNK // CG, group, 0)
    plsc.subcore_barrier()

    pltpu.sync_copy(cnt_sh.at[pl.ds(sid * RPT, RPT)],
                    cnt_hbm.at[cid, pl.ds(sid * RPT, RPT)])


@functools.lru_cache(maxsize=None)
def _make_sc_cnt():
    mesh = plsc.VectorSubcoreMesh(core_axis_name="c", subcore_axis_name="s",
                                  num_cores=NC, num_subcores=NS)
    scratch = [
        pltpu.VMEM((CG, K), jnp.int32),
        pltpu.VMEM((K, CW), jnp.float32),
        pltpu.VMEM_SHARED((NPAD, CW), jnp.float32),
    ]
    return pl.kernel(
        _sc_cnt_body,
        out_type=jax.ShapeDtypeStruct((NC, NPAD, CW), jnp.float32),
        mesh=mesh,
        scratch_types=scratch,
    )


def _tc_in_body(h_ref, ws_ref, wn_ref, b_ref, s_ref, n_ref):
    x = h_ref[...]
    s_ref[...] = (jnp.dot(x, ws_ref[...], preferred_element_type=jnp.float32)
                  + b_ref[...])
    n_ref[...] = jnp.dot(x, wn_ref[...], preferred_element_type=jnp.float32)


def _tc_mid_body(s_ref, acc_ref, cnt_ref, g_ref, bn_ref, ws_ref, wn_ref,
                 b_ref, sn_ref, nn_ref):
    deg = cnt_ref[0, :, 0:1] + cnt_ref[1, :, 0:1]
    inv = 1.0 / jnp.maximum(deg, 1.0)
    x = s_ref[...] + (acc_ref[0] + acc_ref[1]) * inv
    mu = jnp.mean(x, axis=-1, keepdims=True)
    var = jnp.mean((x - mu) ** 2, axis=-1, keepdims=True)
    x = (x - mu) * lax.rsqrt(var + 1e-5) * g_ref[...] + bn_ref[...]
    x = jnp.maximum(x, 0.0)
    sn_ref[...] = (jnp.dot(x, ws_ref[...], preferred_element_type=jnp.float32)
                   + b_ref[...])
    nn_ref[...] = jnp.dot(x, wn_ref[...], preferred_element_type=jnp.float32)


def _tc_out_body(s_ref, acc_ref, cnt_ref, o_ref):
    deg = cnt_ref[0, :, 0:1] + cnt_ref[1, :, 0:1]
    inv = 1.0 / jnp.maximum(deg, 1.0)
    o_ref[...] = (s_ref[:, :F_OUT]
                  + (acc_ref[0, :, :F_OUT] + acc_ref[1, :, :F_OUT]) * inv)


_R = 1000


def _full(shape):
    return pl.BlockSpec(shape, lambda i: (0,) * len(shape))


@functools.lru_cache(maxsize=None)
def _make_tc_in(F, FO):
    return pl.pallas_call(
        _tc_in_body,
        grid=(N // _R,),
        in_specs=[
            pl.BlockSpec((_R, F), lambda i: (i, 0)),
            _full((F, FO)), _full((F, FO)), _full((FO,)),
        ],
        out_specs=[pl.BlockSpec((_R, FO), lambda i: (i, 0))] * 2,
        out_shape=[jax.ShapeDtypeStruct((N, FO), jnp.float32)] * 2,
    )


@functools.lru_cache(maxsize=None)
def _make_tc_mid(F, FO):
    return pl.pallas_call(
        _tc_mid_body,
        grid=(N // _R,),
        in_specs=[
            pl.BlockSpec((_R, F), lambda i: (i, 0)),
            pl.BlockSpec((NC, _R, F), lambda i: (0, i, 0)),
            pl.BlockSpec((NC, _R, CW), lambda i: (0, i, 0)),
            _full((F,)), _full((F,)),
            _full((F, FO)), _full((F, FO)), _full((FO,)),
        ],
        out_specs=[pl.BlockSpec((_R, FO), lambda i: (i, 0))] * 2,
        out_shape=[jax.ShapeDtypeStruct((N, FO), jnp.float32)] * 2,
    )


@functools.lru_cache(maxsize=None)
def _make_tc_out():
    return pl.pallas_call(
        _tc_out_body,
        grid=(N // _R,),
        in_specs=[
            pl.BlockSpec((_R, F_HID), lambda i: (i, 0)),
            pl.BlockSpec((NC, _R, F_HID), lambda i: (0, i, 0)),
            pl.BlockSpec((NC, _R, CW), lambda i: (0, i, 0)),
        ],
        out_specs=pl.BlockSpec((_R, F_OUT), lambda i: (i, 0)),
        out_shape=jax.ShapeDtypeStruct((N, F_OUT), jnp.float32),
    )


def kernel(h, edge_index, W_self1, W_neigh1, b1, ln1_g, ln1_b,
           W_self2, W_neigh2, b2, ln2_g, ln2_b, W_self3, W_neigh3, b3):
    src = edge_index[0]
    dst = edge_index[1]
    e_per_tile = src.shape[0] // NW
    padt = NCHUNK * K - e_per_tile
    ndummy = NPAD - N
    psrc = (jnp.arange(NW * padt, dtype=jnp.int32) % N).reshape(NW, padt)
    src3 = jnp.concatenate(
        [src.reshape(NW, e_per_tile), psrc], axis=1).reshape(NW, NCHUNK, K)
    dmy = (N + (jnp.arange(NW * padt, dtype=jnp.int32) % ndummy)
           ).reshape(NW, padt)
    dst3 = jnp.concatenate(
        [dst.reshape(NW, e_per_tile), dmy], axis=1).reshape(NW, NCHUNK, K)

    zeros_acc = jnp.zeros((NPAD, F_HID), jnp.float32)
    zeros_cnt = jnp.zeros((NPAD, CW), jnp.float32)
    ones_cnt = jnp.ones((K, CW), jnp.float32)

    pad3 = F_HID - F_OUT
    W_self3p = jnp.pad(W_self3, ((0, 0), (0, pad3)))
    W_neigh3p = jnp.pad(W_neigh3, ((0, 0), (0, pad3)))
    b3p = jnp.pad(b3, (0, pad3))

    s1, n1 = _make_tc_in(F_IN, F_HID)(h, W_self1, W_neigh1, b1)
    cnt = _make_sc_cnt()(dst3, ones_cnt, zeros_cnt)
    tok = (cnt[0, 0, 0] * 0.0).astype(jnp.int32)
    acc1 = _make_sc_agg(F_HID)(n1, src3 + tok, dst3, zeros_acc)
    s2, n2 = _make_tc_mid(F_HID, F_HID)(
        s1, acc1, cnt, ln1_g, ln1_b, W_self2, W_neigh2, b2)
    acc2 = _make_sc_agg(F_HID)(n2, src3, dst3, zeros_acc)
    s3, n3 = _make_tc_mid(F_HID, F_HID)(
        s2, acc2, cnt, ln2_g, ln2_b, W_self3p, W_neigh3p, b3p)
    acc3 = _make_sc_agg(F_HID)(n3, src3, dst3, zeros_acc)
    out = _make_tc_out()(s3, acc3, cnt)
    return out
